# baseline (speedup 1.0000x reference)
import dataclasses
import functools

import jax
import jax.numpy as jnp
from jax import lax
from jax.experimental import pallas as pl
from jax.experimental.pallas import tpu as pltpu
from jax.experimental.pallas import tpu_sc as plsc

B = 4096
N = 200
C = 64
R = N * B

NC = 2
NS = 16
NW = NC * NS
ROWS_PER_W = R // NW
CHUNK = 128
CHUNKS_PER_W = ROWS_PER_W // CHUNK
BBLKS = B // CHUNK


def _sc_compiler_params():
    cp = pltpu.CompilerParams(use_tc_tiling_on_sc=False)
    if "needs_layout_passes" in pltpu.CompilerParams.__dataclass_fields__:
        cp = dataclasses.replace(cp, needs_layout_passes=False)
    return cp


def _sc_gather_sum(idx_s, idx_c, shape_table, color_table):
    mesh = plsc.VectorSubcoreMesh(core_axis_name="c", subcore_axis_name="s")

    @functools.partial(
        pl.kernel,
        out_type=jax.ShapeDtypeStruct((N * 8, BBLKS, 8, CHUNK), jnp.float32),
        mesh=mesh,
        scratch_types=[
            pltpu.VMEM((ROWS_PER_W,), jnp.int32),
            pltpu.VMEM((ROWS_PER_W,), jnp.int32),
            pltpu.VMEM((2, CHUNK, C), jnp.float32),
            pltpu.VMEM((2, CHUNK, C), jnp.float32),
            pltpu.VMEM((2, 8, 8, CHUNK), jnp.float32),
            pltpu.SemaphoreType.DMA,
            pltpu.SemaphoreType.DMA,
            pltpu.SemaphoreType.DMA,
            pltpu.SemaphoreType.DMA,
        ],
        compiler_params=_sc_compiler_params(),
    )
    def k(idx_s_hbm, idx_c_hbm, stab_hbm, ctab_hbm, out_hbm,
          idxs_v, idxc_v, rows_s, rows_c, out_v, gs0, gs1, ws0, ws1):
        gsem = (gs0, gs1)
        wsem = (ws0, ws1)
        wid = lax.axis_index("s") * NC + lax.axis_index("c")
        base = wid * ROWS_PER_W
        cbase = wid * CHUNKS_PER_W

        pltpu.sync_copy(idx_s_hbm.at[pl.ds(base, ROWS_PER_W)], idxs_v)
        pltpu.sync_copy(idx_c_hbm.at[pl.ds(base, ROWS_PER_W)], idxc_v)

        def fire(t, p):
            isl = idxs_v.at[pl.ds(t * CHUNK, CHUNK)]
            icl = idxc_v.at[pl.ds(t * CHUNK, CHUNK)]
            pltpu.async_copy(stab_hbm.at[isl], rows_s.at[p], gsem[p])
            pltpu.async_copy(ctab_hbm.at[icl], rows_c.at[p], gsem[p])

        def drain_gather(p):
            pltpu.make_async_copy(stab_hbm.at[pl.ds(0, CHUNK)],
                                  rows_s.at[p], gsem[p]).wait()
            pltpu.make_async_copy(ctab_hbm.at[pl.ds(0, CHUNK)],
                                  rows_c.at[p], gsem[p]).wait()

        def drain_write(p):
            pltpu.make_async_copy(out_hbm.at[pl.ds(0, 8), 0],
                                  out_v.at[p], wsem[p]).wait()

        iota = lax.iota(jnp.int32, 16)

        fire(0, 0)

        @pl.loop(0, CHUNKS_PER_W // 2)
        def _(g):
            for p in (0, 1):
                t = g * 2 + p
                gt = cbase + t
                n8 = (gt >> 5) * 8
                bb = gt & (BBLKS - 1)

                @pl.when(t < CHUNKS_PER_W - 1)
                def _():
                    fire(t + 1, 1 - p)

                drain_gather(p)

                @pl.when(t >= 2)
                def _():
                    drain_write(p)

                @pl.loop(0, CHUNK)
                def _(i):
                    blv = jnp.full((16,), i, jnp.int32)
                    for j in range(C // 16):
                        cv = iota + (16 * j)
                        x = (rows_s[p, i, pl.ds(16 * j, 16)]
                             + rows_c[p, i, pl.ds(16 * j, 16)])
                        plsc.store_scatter(
                            out_v.at[p],
                            [lax.shift_right_logical(cv, 3),
                             lax.bitwise_and(cv, 7), blv], x)

                pltpu.async_copy(out_v.at[p],
                                 out_hbm.at[pl.ds(n8, 8), bb], wsem[p])

        drain_write(0)
        drain_write(1)

    return k(idx_s, idx_c, shape_table, color_table)


def _tc_pose_add(pose_t, sum4d, Wt, b2d):
    NBLK = 4

    def body(pose_ref, sum_ref, wt_ref, b_ref, out_ref):
        for nn in range(NBLK):
            mm = jnp.dot(wt_ref[...], pose_ref[nn],
                         preferred_element_type=jnp.float32) + b_ref[...]
            for t in range(8):
                st = jnp.transpose(sum_ref[nn * 8 + t],
                                   (1, 0, 2)).reshape(8, B)
                out_ref[nn, pl.ds(8 * t, 8), :] = mm[8 * t:8 * t + 8, :] + st

    return pl.pallas_call(
        body,
        grid=(N // NBLK,),
        in_specs=[
            pl.BlockSpec((NBLK, 16, B), lambda i: (i, 0, 0)),
            pl.BlockSpec((NBLK * 8, BBLKS, 8, CHUNK), lambda i: (i, 0, 0, 0)),
            pl.BlockSpec((C, 16), lambda i: (0, 0)),
            pl.BlockSpec((C, 1), lambda i: (0, 0)),
        ],
        out_specs=pl.BlockSpec((NBLK, C, B), lambda i: (i, 0, 0)),
        out_shape=jax.ShapeDtypeStruct((N, C, B), jnp.float32),
    )(pose_t, sum4d, Wt, b2d)


def kernel(shape, color, pose, shape_table, color_table, W, b):
    idx_s = shape.astype(jnp.int32).T.reshape(R)
    idx_c = color.astype(jnp.int32).T.reshape(R)
    sum4d = _sc_gather_sum(idx_s, idx_c, shape_table, color_table)

    pose_t = pose.transpose(1, 2, 0)
    out_cb = _tc_pose_add(pose_t, sum4d, W.T, b.reshape(C, 1))
    return out_cb.transpose(0, 2, 1)


# speedup vs baseline: 1.7702x; 1.7702x over previous
import dataclasses
import functools

import jax
import jax.numpy as jnp
from jax import lax
from jax.experimental import pallas as pl
from jax.experimental.pallas import tpu as pltpu
from jax.experimental.pallas import tpu_sc as plsc

B = 4096
N = 200
C = 64
R = N * B

NC = 2
NS = 16
NW = NC * NS
ROWS_PER_W = R // NW
CHUNK = 128
CHUNKS_PER_W = ROWS_PER_W // CHUNK
BBLKS = B // CHUNK


def _sc_compiler_params():
    cp = pltpu.CompilerParams(use_tc_tiling_on_sc=False)
    if "needs_layout_passes" in pltpu.CompilerParams.__dataclass_fields__:
        cp = dataclasses.replace(cp, needs_layout_passes=False)
    return cp


def _sc_gather_sum(idx_s, idx_c, shape_table, color_table):
    mesh = plsc.VectorSubcoreMesh(core_axis_name="c", subcore_axis_name="s")

    @functools.partial(
        pl.kernel,
        out_type=jax.ShapeDtypeStruct((N * 8, BBLKS, 8, CHUNK), jnp.float32),
        mesh=mesh,
        scratch_types=[
            pltpu.VMEM((ROWS_PER_W,), jnp.int32),
            pltpu.VMEM((ROWS_PER_W,), jnp.int32),
            pltpu.VMEM((2, CHUNK, C), jnp.float32),
            pltpu.VMEM((2, CHUNK, C), jnp.float32),
            pltpu.VMEM((2, 8, 8, CHUNK + 1), jnp.float32),
            pltpu.SemaphoreType.DMA,
            pltpu.SemaphoreType.DMA,
            pltpu.SemaphoreType.DMA,
            pltpu.SemaphoreType.DMA,
        ],
        compiler_params=_sc_compiler_params(),
    )
    def k(idx_s_hbm, idx_c_hbm, stab_hbm, ctab_hbm, out_hbm,
          idxs_v, idxc_v, rows_s, rows_c, out_v, gs0, gs1, ws0, ws1):
        gsem = (gs0, gs1)
        wsem = (ws0, ws1)
        wid = lax.axis_index("s") * NC + lax.axis_index("c")
        base = wid * ROWS_PER_W
        cbase = wid * CHUNKS_PER_W

        pltpu.sync_copy(idx_s_hbm.at[pl.ds(base, ROWS_PER_W)], idxs_v)
        pltpu.sync_copy(idx_c_hbm.at[pl.ds(base, ROWS_PER_W)], idxc_v)

        def fire(t, p):
            isl = idxs_v.at[pl.ds(t * CHUNK, CHUNK)]
            icl = idxc_v.at[pl.ds(t * CHUNK, CHUNK)]
            pltpu.async_copy(stab_hbm.at[isl], rows_s.at[p], gsem[p])
            pltpu.async_copy(ctab_hbm.at[icl], rows_c.at[p], gsem[p])

        def drain_gather(p):
            pltpu.make_async_copy(stab_hbm.at[pl.ds(0, CHUNK)],
                                  rows_s.at[p], gsem[p]).wait()
            pltpu.make_async_copy(ctab_hbm.at[pl.ds(0, CHUNK)],
                                  rows_c.at[p], gsem[p]).wait()

        def out_src(p):
            return out_v.at[p, :, :, pl.ds(0, CHUNK)]

        def drain_write(p):
            pltpu.make_async_copy(out_hbm.at[pl.ds(0, 8), 0],
                                  out_src(p), wsem[p]).wait()

        iota = lax.iota(jnp.int32, 16)

        fire(0, 0)

        @pl.loop(0, CHUNKS_PER_W // 2)
        def _(g):
            for p in (0, 1):
                t = g * 2 + p
                gt = cbase + t
                n8 = (gt >> 5) * 8
                bb = gt & (BBLKS - 1)

                @pl.when(t < CHUNKS_PER_W - 1)
                def _():
                    fire(t + 1, 1 - p)

                drain_gather(p)

                @pl.when(t >= 2)
                def _():
                    drain_write(p)

                @pl.loop(0, CHUNK // 8)
                def _(i0):
                    for q in range(8):
                        i = i0 * 8 + q
                        blv = jnp.full((16,), i, jnp.int32)
                        for j in range(C // 16):
                            cv = iota + (16 * j)
                            x = (rows_s[p, i, pl.ds(16 * j, 16)]
                                 + rows_c[p, i, pl.ds(16 * j, 16)])
                            plsc.store_scatter(
                                out_v.at[p],
                                [lax.shift_right_logical(cv, 3),
                                 lax.bitwise_and(cv, 7), blv], x)

                pltpu.async_copy(out_src(p),
                                 out_hbm.at[pl.ds(n8, 8), bb], wsem[p])

        drain_write(0)
        drain_write(1)

    return k(idx_s, idx_c, shape_table, color_table)


def _tc_pose_add(pose_t, sum4d, Wt, b2d):
    NBLK = 4

    def body(pose_ref, sum_ref, wt_ref, b_ref, out_ref):
        for nn in range(NBLK):
            mm = jnp.dot(wt_ref[...], pose_ref[nn],
                         preferred_element_type=jnp.float32) + b_ref[...]
            for t in range(8):
                st = jnp.transpose(sum_ref[nn * 8 + t],
                                   (1, 0, 2)).reshape(8, B)
                out_ref[nn, pl.ds(8 * t, 8), :] = mm[8 * t:8 * t + 8, :] + st

    return pl.pallas_call(
        body,
        grid=(N // NBLK,),
        in_specs=[
            pl.BlockSpec((NBLK, 16, B), lambda i: (i, 0, 0)),
            pl.BlockSpec((NBLK * 8, BBLKS, 8, CHUNK), lambda i: (i, 0, 0, 0)),
            pl.BlockSpec((C, 16), lambda i: (0, 0)),
            pl.BlockSpec((C, 1), lambda i: (0, 0)),
        ],
        out_specs=pl.BlockSpec((NBLK, C, B), lambda i: (i, 0, 0)),
        out_shape=jax.ShapeDtypeStruct((N, C, B), jnp.float32),
    )(pose_t, sum4d, Wt, b2d)


def kernel(shape, color, pose, shape_table, color_table, W, b):
    idx_s = shape.astype(jnp.int32).T.reshape(R)
    idx_c = color.astype(jnp.int32).T.reshape(R)
    sum4d = _sc_gather_sum(idx_s, idx_c, shape_table, color_table)

    pose_t = pose.transpose(1, 2, 0)
    out_cb = _tc_pose_add(pose_t, sum4d, W.T, b.reshape(C, 1))
    return out_cb.transpose(0, 2, 1)
